# baseline (speedup 1.0000x reference)
import functools

import jax
import jax.numpy as jnp
from jax import lax
from jax.experimental import pallas as pl
from jax.experimental.pallas import tpu as pltpu
from jax.experimental.pallas import tpu_sc as plsc

NC = 2
NS = 16
L = 16
NW = NC * NS

_MESH = plsc.VectorSubcoreMesh(
    core_axis_name="c", subcore_axis_name="s", num_cores=NC, num_subcores=NS)
_SC_PARAMS = pltpu.CompilerParams(
    needs_layout_passes=False, use_tc_tiling_on_sc=False)


def _zero2d(ref):
  z = jnp.zeros((L,), ref.dtype)
  for r in range(ref.shape[0]):
    ref[r] = z


def _wid():
  return lax.axis_index("s") * NC + lax.axis_index("c")


def _label_pack_kernel(edge_index1, x1_batch, *, n1, e1, n_tiles):
  per_tile = e1 // n_tiles
  words_pt = per_tile // 4
  n_groups = words_pt // L

  @functools.partial(
      pl.kernel,
      out_type=[
          jax.ShapeDtypeStruct((e1 // 4,), jnp.int32),
          jax.ShapeDtypeStruct((n_tiles, L, L), jnp.int32),
      ],
      mesh=_MESH,
      scratch_types=[
          pltpu.VMEM((n1,), jnp.int32),
          pltpu.VMEM((per_tile,), jnp.int32),
          pltpu.VMEM((words_pt,), jnp.int32),
          pltpu.VMEM((L, L), jnp.int32),
      ],
      compiler_params=_SC_PARAMS,
  )
  def body(ei1_hbm, x1b_hbm, packed_hbm, nh_hbm, x1b_v, d1_v, pk_v, nh_v):
    wid = _wid()

    @pl.when(wid < n_tiles)
    def _():
      pltpu.sync_copy(x1b_hbm, x1b_v)
      pltpu.sync_copy(ei1_hbm.at[1, pl.ds(wid * per_tile, per_tile)], d1_v)
      _zero2d(nh_v)
      iota = lax.iota(jnp.int32, L)
      ones = jnp.ones((L,), jnp.int32)

      def grp(w0, carry):
        acc = jnp.zeros((L,), jnp.int32)
        for p in range(4):
          e_idx = (w0 * L + iota) * 4 + p
          node = plsc.load_gather(d1_v, [e_idx])
          lab = plsc.load_gather(x1b_v, [node])
          plsc.addupdate_scatter(nh_v, [lab, iota], ones)
          acc = acc | (lab << (8 * p))
        pk_v[pl.ds(w0 * L, L)] = acc
        return carry

      lax.fori_loop(0, n_groups, grp, 0)
      pltpu.sync_copy(pk_v, packed_hbm.at[pl.ds(wid * words_pt, words_pt)])
      pltpu.sync_copy(nh_v, nh_hbm.at[wid])

  return body(edge_index1, x1_batch)


def _accum_kernel(edge_index2, packed, edge_attr2, *, e2, d2, chunk, sub):
  per_tile = e2 // NW
  n_chunks = per_tile // chunk
  n_sub = chunk // sub
  g_per_sub = sub // L
  n_words = packed.shape[0]

  assert n_chunks % 2 == 0

  @functools.partial(
      pl.kernel,
      out_type=[
          jax.ShapeDtypeStruct((NC, L, d2), jnp.float32),
          jax.ShapeDtypeStruct((NW, L, L), jnp.int32),
      ],
      mesh=_MESH,
      scratch_types=[
          pltpu.VMEM((n_words,), jnp.int32),
          pltpu.VMEM((2, chunk), jnp.int32),
          pltpu.VMEM((2, chunk, d2), jnp.float32),
          pltpu.VMEM((2, n_sub, sub), jnp.int32),
          pltpu.VMEM((L, L), jnp.int32),
          pltpu.VMEM((L, d2), jnp.float32),
          pltpu.VMEM_SHARED((L, d2), jnp.float32),
          pltpu.SemaphoreType.DMA,
          pltpu.SemaphoreType.DMA,
          pltpu.SemaphoreType.DMA,
          pltpu.SemaphoreType.DMA,
      ],
      compiler_params=_SC_PARAMS,
  )
  def body(ei2_hbm, packed_hbm, attr_hbm, a_hbm, mh_hbm,
           pk_v, d2_v, rows_v, lbl_v, mh_v, zero_v, a_sh,
           sem_in0, sem_in1, sem_sc0, sem_sc1):
    cid = lax.axis_index("c")
    sid = lax.axis_index("s")
    wid = sid * NC + cid
    base = wid * per_tile
    sem_in = (sem_in0, sem_in1)
    sem_sc = (sem_sc0, sem_sc1)

    def start_in(c, b):
      off = base + c * chunk
      pltpu.async_copy(ei2_hbm.at[1, pl.ds(off, chunk)], d2_v.at[b], sem_in[b])
      pltpu.async_copy(attr_hbm.at[pl.ds(off, chunk)], rows_v.at[b], sem_in[b])

    def wait_in(b):
      pltpu.make_async_copy(ei2_hbm.at[1, pl.ds(0, chunk)], d2_v.at[b],
                            sem_in[b]).wait()
      pltpu.make_async_copy(attr_hbm.at[pl.ds(0, chunk)], rows_v.at[b],
                            sem_in[b]).wait()

    def fire_scatters(b):
      for j in range(n_sub):
        pltpu.async_copy(rows_v.at[b, pl.ds(j * sub, sub)],
                         a_sh.at[lbl_v.at[b, j]], sem_sc[b], add=True)

    def drain_scatters(b):
      for j in range(n_sub):
        pltpu.make_async_copy(rows_v.at[b, pl.ds(j * sub, sub)],
                              a_sh.at[lbl_v.at[b, j]], sem_sc[b]).wait()

    pltpu.sync_copy(packed_hbm, pk_v)
    _zero2d(mh_v)

    @pl.when(sid == 0)
    def _():
      _zero2d(zero_v)
      pltpu.sync_copy(zero_v, a_sh)

    plsc.subcore_barrier()
    iota = lax.iota(jnp.int32, L)
    ones = jnp.ones((L,), jnp.int32)
    start_in(0, 0)

    def decode(b):
      def grp(s, carry2):
        for k in range(g_per_sub):
          e1 = d2_v[b, pl.ds(s * sub + k * L, L)]
          word = plsc.load_gather(pk_v, [e1 >> 2])
          lab = (word >> ((e1 & 3) << 3)) & 0xFF
          plsc.addupdate_scatter(mh_v, [lab, iota], ones)
          lbl_v[b, s, pl.ds(k * L, L)] = lab
        return carry2

      lax.fori_loop(0, n_sub, grp, 0)

    def pair(i, carry):
      wait_in(0)
      decode(0)

      @pl.when(i > 0)
      def _():
        drain_scatters(1)

      start_in(2 * i + 1, 1)
      fire_scatters(0)
      wait_in(1)
      decode(1)
      drain_scatters(0)

      @pl.when(i < n_chunks // 2 - 1)
      def _():
        start_in(2 * i + 2, 0)

      fire_scatters(1)
      return carry

    lax.fori_loop(0, n_chunks // 2, pair, 0)
    drain_scatters(1)
    plsc.subcore_barrier()

    @pl.when(sid == 0)
    def _():
      pltpu.sync_copy(a_sh, a_hbm.at[cid])

    pltpu.sync_copy(mh_v, mh_hbm.at[wid])

  return body(edge_index2, packed, edge_attr2)


def _epilogue(a_part, nh, mh, w1, b1, w2, b2, *, b, d1, d2):
  def body(a_ref, nh_ref, mh_ref, w1_ref, b1_ref, w2_ref, b2_ref, out_ref):
    a = jnp.sum(a_ref[...], axis=0)
    n = jnp.sum(nh_ref[...], axis=(0, 2)).astype(jnp.float32)
    m = jnp.sum(mh_ref[...], axis=(0, 2)).astype(jnp.float32)
    t = (jnp.dot(a, w2_ref[...], preferred_element_type=jnp.float32)
         + m[:, None] * b2_ref[...])
    out = (jnp.dot(t, w1_ref[...], preferred_element_type=jnp.float32)
           + n[:, None] * b1_ref[...])
    out_ref[...] = out * 0.5

  return pl.pallas_call(
      body,
      out_shape=jax.ShapeDtypeStruct((b, d1), jnp.float32),
  )(a_part, nh, mh, w1, b1.reshape(1, d1), w2, b2.reshape(1, d2))


def kernel(x1, x2, edge_index1, edge_index2, x1_batch, x2_batch,
           edge_attr1, edge_attr2, W_mlp, b_mlp, W_mlp2, b_mlp2):
  n1 = x1.shape[0]
  e1 = x2.shape[0]
  e2 = edge_attr2.shape[0]
  d2 = edge_attr2.shape[1]
  d1 = W_mlp.shape[1]
  b = 16

  packed, nh = _label_pack_kernel(edge_index1, x1_batch, n1=n1, e1=e1,
                                  n_tiles=20)
  a_part, mh = _accum_kernel(edge_index2, packed, edge_attr2, e2=e2, d2=d2,
                             chunk=800, sub=80)
  return _epilogue(a_part, nh, mh, W_mlp, b_mlp, W_mlp2, b_mlp2,
                   b=b, d1=d1, d2=d2)


# speedup vs baseline: 1.6255x; 1.6255x over previous
import functools

import jax
import jax.numpy as jnp
from jax import lax
from jax.experimental import pallas as pl
from jax.experimental.pallas import tpu as pltpu
from jax.experimental.pallas import tpu_sc as plsc

NC = 2
NS = 16
L = 16
NW = NC * NS

_MESH = plsc.VectorSubcoreMesh(
    core_axis_name="c", subcore_axis_name="s", num_cores=NC, num_subcores=NS)
_SC_PARAMS = pltpu.CompilerParams(
    needs_layout_passes=False, use_tc_tiling_on_sc=False)


def _zero2d(ref):
  z = jnp.zeros((L,), ref.dtype)
  for r in range(ref.shape[0]):
    ref[r] = z


def _wid():
  return lax.axis_index("s") * NC + lax.axis_index("c")


def _label_pack_kernel(edge_index1, x1_batch, *, n1, e1, n_tiles):
  per_tile = e1 // n_tiles
  words_pt = per_tile // 4
  n_groups = words_pt // L

  @functools.partial(
      pl.kernel,
      out_type=[
          jax.ShapeDtypeStruct((e1 // 4,), jnp.int32),
          jax.ShapeDtypeStruct((n_tiles, L, L), jnp.int32),
      ],
      mesh=_MESH,
      scratch_types=[
          pltpu.VMEM((n1,), jnp.int32),
          pltpu.VMEM((per_tile,), jnp.int32),
          pltpu.VMEM((words_pt,), jnp.int32),
          pltpu.VMEM((L, L), jnp.int32),
      ],
      compiler_params=_SC_PARAMS,
  )
  def body(ei1_hbm, x1b_hbm, packed_hbm, nh_hbm, x1b_v, d1_v, pk_v, nh_v):
    wid = _wid()

    @pl.when(wid < n_tiles)
    def _():
      pltpu.sync_copy(x1b_hbm, x1b_v)
      pltpu.sync_copy(ei1_hbm.at[1, pl.ds(wid * per_tile, per_tile)], d1_v)
      _zero2d(nh_v)
      iota = lax.iota(jnp.int32, L)
      ones = jnp.ones((L,), jnp.int32)

      def grp(w0, carry):
        acc = jnp.zeros((L,), jnp.int32)
        for p in range(4):
          e_idx = (w0 * L + iota) * 4 + p
          node = plsc.load_gather(d1_v, [e_idx])
          lab = plsc.load_gather(x1b_v, [node])
          plsc.addupdate_scatter(nh_v, [lab, iota], ones)
          acc = acc | (lab << (8 * p))
        pk_v[pl.ds(w0 * L, L)] = acc
        return carry

      lax.fori_loop(0, n_groups, grp, 0)
      pltpu.sync_copy(pk_v, packed_hbm.at[pl.ds(wid * words_pt, words_pt)])
      pltpu.sync_copy(nh_v, nh_hbm.at[wid])

  return body(edge_index1, x1_batch)


def _accum_kernel(edge_index2, packed, attr_t, *, e2, d2, chunk):
  per_tile = e2 // NW
  n_chunks = per_tile // chunk
  n_groups = chunk // L
  n_words = packed.shape[0]

  assert n_chunks % 2 == 0

  @functools.partial(
      pl.kernel,
      out_type=[
          jax.ShapeDtypeStruct((NW, d2, L), jnp.float32),
          jax.ShapeDtypeStruct((NW, L, L), jnp.int32),
      ],
      mesh=_MESH,
      scratch_types=[
          pltpu.VMEM((n_words,), jnp.int32),
          pltpu.VMEM((2, chunk), jnp.int32),
          pltpu.VMEM((2, d2, chunk), jnp.float32),
          pltpu.VMEM((d2, L), jnp.float32),
          pltpu.VMEM((L, L), jnp.int32),
          pltpu.SemaphoreType.DMA,
          pltpu.SemaphoreType.DMA,
      ],
      compiler_params=_SC_PARAMS,
  )
  def body(ei2_hbm, packed_hbm, attr_hbm, a_hbm, mh_hbm,
           pk_v, d2_v, rows_v, a_v, mh_v, sem_in0, sem_in1):
    cid = lax.axis_index("c")
    sid = lax.axis_index("s")
    wid = sid * NC + cid
    base = wid * per_tile
    sem_in = (sem_in0, sem_in1)

    def start_in(c, b):
      off = base + c * chunk
      pltpu.async_copy(ei2_hbm.at[1, pl.ds(off, chunk)], d2_v.at[b], sem_in[b])
      pltpu.async_copy(attr_hbm.at[:, pl.ds(off, chunk)], rows_v.at[b],
                       sem_in[b])

    def wait_in(b):
      pltpu.make_async_copy(ei2_hbm.at[1, pl.ds(0, chunk)], d2_v.at[b],
                            sem_in[b]).wait()
      pltpu.make_async_copy(attr_hbm.at[:, pl.ds(0, chunk)], rows_v.at[b],
                            sem_in[b]).wait()

    pltpu.sync_copy(packed_hbm, pk_v)
    _zero2d(a_v)
    _zero2d(mh_v)
    iota = lax.iota(jnp.int32, L)
    ones = jnp.ones((L,), jnp.int32)
    start_in(0, 0)

    def consume(c, b, prefetch):
      wait_in(b)

      @pl.when(prefetch)
      def _():
        start_in(c + 1, 1 - b)

      def grp(g, carry2):
        e1 = d2_v[b, pl.ds(g * L, L)]
        word = plsc.load_gather(pk_v, [e1 >> 2])
        lab = (word >> ((e1 & 3) << 3)) & 0xFF
        plsc.addupdate_scatter(mh_v, [lab, iota], ones)
        for f in range(d2):
          vals = rows_v[b, f, pl.ds(g * L, L)]
          plsc.addupdate_scatter(a_v, [jnp.full((L,), f, jnp.int32), lab],
                                 vals)
        return carry2

      lax.fori_loop(0, n_groups, grp, 0)

    def pair(i, carry):
      consume(2 * i, 0, jnp.bool_(True))
      consume(2 * i + 1, 1, i < n_chunks // 2 - 1)
      return carry

    lax.fori_loop(0, n_chunks // 2, pair, 0)
    pltpu.sync_copy(a_v, a_hbm.at[wid])
    pltpu.sync_copy(mh_v, mh_hbm.at[wid])

  return body(edge_index2, packed, attr_t)


def _epilogue(a_part, nh, mh, w1, b1, w2, b2, *, b, d1, d2):
  def body(a_ref, nh_ref, mh_ref, w1_ref, b1_ref, w2_ref, b2_ref, out_ref):
    a_fb = jnp.sum(a_ref[...], axis=0)
    n = jnp.sum(nh_ref[...], axis=(0, 2)).astype(jnp.float32)
    m = jnp.sum(mh_ref[...], axis=(0, 2)).astype(jnp.float32)
    t = (lax.dot_general(a_fb, w2_ref[...], (((0,), (0,)), ((), ())),
                         preferred_element_type=jnp.float32)
         + m[:, None] * b2_ref[...])
    out = (jnp.dot(t, w1_ref[...], preferred_element_type=jnp.float32)
           + n[:, None] * b1_ref[...])
    out_ref[...] = out * 0.5

  return pl.pallas_call(
      body,
      out_shape=jax.ShapeDtypeStruct((b, d1), jnp.float32),
  )(a_part, nh, mh, w1, b1.reshape(1, d1), w2, b2.reshape(1, d2))


def kernel(x1, x2, edge_index1, edge_index2, x1_batch, x2_batch,
           edge_attr1, edge_attr2, W_mlp, b_mlp, W_mlp2, b_mlp2):
  n1 = x1.shape[0]
  e1 = x2.shape[0]
  e2 = edge_attr2.shape[0]
  d2 = edge_attr2.shape[1]
  d1 = W_mlp.shape[1]
  b = 16

  packed, nh = _label_pack_kernel(edge_index1, x1_batch, n1=n1, e1=e1,
                                  n_tiles=20)
  a_part, mh = _accum_kernel(edge_index2, packed, edge_attr2.T, e2=e2, d2=d2,
                             chunk=800)
  return _epilogue(a_part, nh, mh, W_mlp, b_mlp, W_mlp2, b_mlp2,
                   b=b, d1=d1, d2=d2)
